# Initial kernel scaffold; baseline (speedup 1.0000x reference)
import jax
import jax.numpy as jnp
from jax.experimental import pallas as pl


def kernel(source_types, length_bins, src_emb, len_emb):
    raise NotImplementedError("write your pallas kernel here")


# speedup vs baseline: 1.8264x; 1.8264x over previous
import functools

import jax
import jax.numpy as jnp
from jax import lax
from jax.experimental import pallas as pl
from jax.experimental.pallas import tpu as pltpu
from jax.experimental.pallas import tpu_sc as plsc

B = 16384
K = 128
SRC_VOCAB = 5
LEN_VOCAB = 3

NC = 2
NS = 16
L = 16
NW = NC * NS
BPW = B // NW
CHUNK = 128
NCHUNK = BPW // CHUNK

_mesh = plsc.VectorSubcoreMesh(core_axis_name="c", subcore_axis_name="s")


@functools.partial(
    pl.kernel,
    mesh=_mesh,
    out_type=jax.ShapeDtypeStruct((B, K), jnp.float32),
    scratch_types=[
        pltpu.VMEM((BPW,), jnp.int32),
        pltpu.VMEM((BPW,), jnp.int32),
        pltpu.VMEM((NCHUNK, CHUNK), jnp.int32),
        pltpu.VMEM((BPW, K), jnp.float32),
        pltpu.SemaphoreType.DMA,
    ],
)
def _fused_lookup(src_hbm, len_hbm, table_hbm, out_hbm, sidx, lidx, fidx, rows, sem):
    wid = lax.axis_index("s") * NC + lax.axis_index("c")
    base = wid * BPW

    pltpu.sync_copy(src_hbm.at[pl.ds(base, BPW)], sidx)
    pltpu.sync_copy(len_hbm.at[pl.ds(base, BPW)], lidx)

    for k in range(BPW // L):
        j, c = divmod(k, CHUNK // L)
        s = sidx[pl.ds(k * L, L)]
        t = lidx[pl.ds(k * L, L)]
        fidx[j, pl.ds(c * L, L)] = s * LEN_VOCAB + t

    copies = [
        pltpu.async_copy(
            table_hbm.at[fidx.at[j]],
            rows.at[pl.ds(j * CHUNK, CHUNK)],
            sem,
        )
        for j in range(NCHUNK)
    ]
    for cp in copies:
        cp.wait()

    pltpu.sync_copy(rows, out_hbm.at[pl.ds(base, BPW)])


def kernel(source_types, length_bins, src_emb, len_emb):
    k_src = src_emb.shape[1]
    k_len = len_emb.shape[1]
    table = jnp.concatenate(
        [
            jnp.broadcast_to(src_emb[:, None, :], (SRC_VOCAB, LEN_VOCAB, k_src)),
            jnp.broadcast_to(len_emb[None, :, :], (SRC_VOCAB, LEN_VOCAB, k_len)),
        ],
        axis=-1,
    ).reshape(SRC_VOCAB * LEN_VOCAB, K)
    return _fused_lookup(
        source_types.astype(jnp.int32),
        length_bins.astype(jnp.int32),
        table,
    )


# speedup vs baseline: 5.1867x; 2.8398x over previous
import functools

import jax
import jax.numpy as jnp
from jax import lax
from jax.experimental import pallas as pl
from jax.experimental.pallas import tpu as pltpu
from jax.experimental.pallas import tpu_sc as plsc

B = 16384
K = 128
SRC_VOCAB = 5
LEN_VOCAB = 3

NC = 2
NS = 16
L = 16
NW = NC * NS
BPW = B // NW
CHUNK = 128
NCHUNK = BPW // CHUNK

_mesh = plsc.VectorSubcoreMesh(core_axis_name="c", subcore_axis_name="s")


@functools.partial(
    pl.kernel,
    mesh=_mesh,
    out_type=jax.ShapeDtypeStruct((B, K), jnp.float32),
    scratch_types=[
        pltpu.VMEM((BPW,), jnp.int32),
        pltpu.VMEM((BPW,), jnp.int32),
        pltpu.VMEM((NCHUNK, CHUNK), jnp.int32),
        pltpu.VMEM_SHARED((SRC_VOCAB * LEN_VOCAB, K), jnp.float32),
        pltpu.VMEM((BPW, K), jnp.float32),
        pltpu.SemaphoreType.DMA,
        pltpu.SemaphoreType.DMA,
    ],
)
def _fused_lookup(src_hbm, len_hbm, table_hbm, out_hbm,
                  sidx, lidx, fidx, table_v, rows, gsem, ssem):
    wid = lax.axis_index("s") * NC + lax.axis_index("c")
    base = wid * BPW

    @pl.when(lax.axis_index("s") == 0)
    def _():
        pltpu.sync_copy(table_hbm, table_v)

    pltpu.sync_copy(src_hbm.at[pl.ds(base, BPW)], sidx)
    pltpu.sync_copy(len_hbm.at[pl.ds(base, BPW)], lidx)
    plsc.subcore_barrier()

    for k in range(BPW // L):
        j, c = divmod(k, CHUNK // L)
        s = sidx[pl.ds(k * L, L)]
        t = lidx[pl.ds(k * L, L)]
        fidx[j, pl.ds(c * L, L)] = s * LEN_VOCAB + t

    gathers = [
        pltpu.async_copy(
            table_v.at[fidx.at[j]],
            rows.at[pl.ds(j * CHUNK, CHUNK)],
            gsem,
        )
        for j in range(NCHUNK)
    ]
    stores = []
    for j in range(NCHUNK):
        gathers[j].wait()
        stores.append(
            pltpu.async_copy(
                rows.at[pl.ds(j * CHUNK, CHUNK)],
                out_hbm.at[pl.ds(base + j * CHUNK, CHUNK)],
                ssem,
            )
        )
    for st in stores:
        st.wait()


def kernel(source_types, length_bins, src_emb, len_emb):
    k_src = src_emb.shape[1]
    k_len = len_emb.shape[1]
    table = jnp.concatenate(
        [
            jnp.broadcast_to(src_emb[:, None, :], (SRC_VOCAB, LEN_VOCAB, k_src)),
            jnp.broadcast_to(len_emb[None, :, :], (SRC_VOCAB, LEN_VOCAB, k_len)),
        ],
        axis=-1,
    ).reshape(SRC_VOCAB * LEN_VOCAB, K)
    return _fused_lookup(
        source_types.astype(jnp.int32),
        length_bins.astype(jnp.int32),
        table,
    )


# speedup vs baseline: 5.3158x; 1.0249x over previous
import functools

import jax
import jax.numpy as jnp
from jax import lax
from jax.experimental import pallas as pl
from jax.experimental.pallas import tpu as pltpu
from jax.experimental.pallas import tpu_sc as plsc

B = 16384
K = 128
K_SRC = K // 2
K_LEN = K - K_SRC
SRC_VOCAB = 5
LEN_VOCAB = 3
NROWS = SRC_VOCAB * LEN_VOCAB

NC = 2
NS = 16
L = 16
NW = NC * NS
BPW = B // NW
CHUNK = 128
NCHUNK = BPW // CHUNK

_mesh = plsc.VectorSubcoreMesh(core_axis_name="c", subcore_axis_name="s")


@functools.partial(
    pl.kernel,
    mesh=_mesh,
    out_type=jax.ShapeDtypeStruct((B, K), jnp.float32),
    scratch_types=[
        pltpu.VMEM((BPW,), jnp.int32),
        pltpu.VMEM((BPW,), jnp.int32),
        pltpu.VMEM((NCHUNK, CHUNK), jnp.int32),
        pltpu.VMEM((SRC_VOCAB * K_SRC,), jnp.float32),
        pltpu.VMEM((LEN_VOCAB * K_LEN,), jnp.float32),
        pltpu.VMEM((NROWS, K), jnp.float32),
        pltpu.VMEM_SHARED((NROWS, K), jnp.float32),
        pltpu.VMEM((BPW, K), jnp.float32),
        pltpu.SemaphoreType.DMA,
        pltpu.SemaphoreType.DMA,
        pltpu.SemaphoreType.DMA,
    ],
)
def _fused_lookup(src_hbm, len_hbm, src_emb_hbm, len_emb_hbm, out_hbm,
                  sidx, lidx, fidx, semb, lemb, tbl_l, table_v, rows,
                  gsem, ssem, tsem):
    wid = lax.axis_index("s") * NC + lax.axis_index("c")
    base = wid * BPW

    icp1 = pltpu.async_copy(src_hbm.at[pl.ds(base, BPW)], sidx, gsem)
    icp2 = pltpu.async_copy(len_hbm.at[pl.ds(base, BPW)], lidx, gsem)

    @pl.when(lax.axis_index("s") == 0)
    def _():
        e1 = pltpu.async_copy(src_emb_hbm, semb, tsem)
        e2 = pltpu.async_copy(len_emb_hbm, lemb, tsem)
        e1.wait()
        e2.wait()
        for f in range(NROWS):
            s, t = divmod(f, LEN_VOCAB)
            for c in range(K_SRC // L):
                tbl_l[f, pl.ds(c * L, L)] = semb[pl.ds(s * K_SRC + c * L, L)]
            for c in range(K_LEN // L):
                tbl_l[f, pl.ds(K_SRC + c * L, L)] = lemb[pl.ds(t * K_LEN + c * L, L)]
        pltpu.sync_copy(tbl_l, table_v)

    icp1.wait()
    icp2.wait()

    for k in range(BPW // L):
        j, c = divmod(k, CHUNK // L)
        s = sidx[pl.ds(k * L, L)]
        t = lidx[pl.ds(k * L, L)]
        fidx[j, pl.ds(c * L, L)] = s * LEN_VOCAB + t

    plsc.subcore_barrier()

    gathers = [
        pltpu.async_copy(
            table_v.at[fidx.at[j]],
            rows.at[pl.ds(j * CHUNK, CHUNK)],
            gsem,
        )
        for j in range(NCHUNK)
    ]
    stores = []
    for j in range(NCHUNK):
        gathers[j].wait()
        stores.append(
            pltpu.async_copy(
                rows.at[pl.ds(j * CHUNK, CHUNK)],
                out_hbm.at[pl.ds(base + j * CHUNK, CHUNK)],
                ssem,
            )
        )
    for st in stores:
        st.wait()


def kernel(source_types, length_bins, src_emb, len_emb):
    return _fused_lookup(
        source_types.astype(jnp.int32),
        length_bins.astype(jnp.int32),
        src_emb.reshape(-1),
        len_emb.reshape(-1),
    )
